# Initial kernel scaffold; baseline (speedup 1.0000x reference)
import jax
import jax.numpy as jnp
from jax.experimental import pallas as pl


def kernel(x, gate_w, gate_b, log_temp):
    raise NotImplementedError("write your pallas kernel here")


# speedup vs baseline: 1.2242x; 1.2242x over previous
import functools

import jax
import jax.numpy as jnp
from jax import lax
from jax.experimental import pallas as pl
from jax.experimental.pallas import tpu as pltpu
from jax.experimental.pallas import tpu_sc as plsc

_B, _S, _D = 4, 8192, 768
_K = 1638
_KPAD = 1664
_BLK = 2048
_NBLK = (_B * _S) // _BLK
_NV = _S // 16
_NBINS = 256


def _gate_body(x_ref, w_ref, b_ref, out_ref, sp_ref, ent_ref):
    i = pl.program_id(0)
    lg = lax.dot_general(
        x_ref[...], w_ref[...], (((1,), (0,)), ((), ())),
        preferred_element_type=jnp.float32,
    ) + b_ref[0, 0]
    out_ref[...] = lg
    p = jax.nn.sigmoid(lg)
    ent = -(p * jnp.log(p + 1e-10) + (1.0 - p) * jnp.log(1.0 - p + 1e-10))
    sp = jnp.sum(p)
    en = jnp.sum(ent)

    @pl.when(i == 0)
    def _init():
        sp_ref[0, 0] = sp
        ent_ref[0, 0] = en

    @pl.when(i != 0)
    def _acc():
        sp_ref[0, 0] += sp
        ent_ref[0, 0] += en


def _gate(x2d, w, b):
    return pl.pallas_call(
        _gate_body,
        grid=(_NBLK,),
        in_specs=[
            pl.BlockSpec((_BLK, _D), lambda i: (i, 0)),
            pl.BlockSpec((_D, 1), lambda i: (0, 0)),
            pl.BlockSpec(memory_space=pltpu.SMEM),
        ],
        out_specs=[
            pl.BlockSpec((_BLK, 1), lambda i: (i, 0)),
            pl.BlockSpec(memory_space=pltpu.SMEM),
            pl.BlockSpec(memory_space=pltpu.SMEM),
        ],
        out_shape=[
            jax.ShapeDtypeStruct((_B * _S, 1), jnp.float32),
            jax.ShapeDtypeStruct((1, 1), jnp.float32),
            jax.ShapeDtypeStruct((1, 1), jnp.float32),
        ],
        compiler_params=pltpu.CompilerParams(
            dimension_semantics=("arbitrary",),
        ),
    )(x2d, w, b)


def _select_body(logits_hbm, mask_hbm, idx_hbm, lrow, keys, hist, cbuf, mrow, irow):
    wid = lax.axis_index("s") * 2 + lax.axis_index("c")

    @pl.when(wid < _B)
    def _():
        row = wid
        pltpu.sync_copy(logits_hbm.at[pl.ds(row * _S, _S)], lrow)

        lane = lax.iota(jnp.int32, 16)
        ones16 = jnp.ones((16,), jnp.int32)
        sign = jnp.int32(-2147483648)

        def keybody(i, carry):
            bits = lax.bitcast_convert_type(lrow[pl.ds(i * 16, 16)], jnp.int32)
            key = bits ^ (lax.shift_right_arithmetic(bits, 31) & jnp.int32(0x7FFFFFFF))
            key = key + jnp.where(key == jnp.int32(-1), jnp.int32(1), jnp.int32(0))
            keys[pl.ds(i * 16, 16)] = key
            return carry

        lax.fori_loop(0, _NV, keybody, jnp.int32(0))

        prefix = jnp.int32(0)
        k_rem = jnp.int32(_K)

        for lvl in range(4):
            sh = 24 - 8 * lvl

            def zbody(i, carry):
                hist[pl.ds(i * 16, 16)] = jnp.zeros((16,), jnp.int32)
                return carry

            lax.fori_loop(0, (16 * _NBINS) // 16, zbody, jnp.int32(0))

            def hbody(i, carry):
                key = keys[pl.ds(i * 16, 16)]
                ub = key ^ sign
                binv = lax.shift_right_logical(ub, jnp.int32(sh)) & jnp.int32(_NBINS - 1)
                if lvl == 0:
                    m = jnp.ones((16,), jnp.bool_)
                else:
                    m = lax.shift_right_logical(ub, jnp.int32(sh + 8)) == prefix
                plsc.addupdate_scatter(hist, [lane * _NBINS + binv], ones16, mask=m)
                return carry

            lax.fori_loop(0, _NV, hbody, jnp.int32(0))

            def sbody(i, csum):
                tot = hist[pl.ds(i * 16, 16)]
                for j in range(1, 16):
                    tot = tot + hist[pl.ds(j * _NBINS + i * 16, 16)]
                cbuf[pl.ds(i * 16, 16)] = plsc.cumsum(tot) + csum
                return csum + jnp.sum(tot)

            n_match = lax.fori_loop(0, _NBINS // 16, sbody, jnp.int32(0))

            excess = n_match - k_rem

            def bbody(i, acc):
                c = cbuf[pl.ds(i * 16, 16)]
                return acc + jnp.where(c <= excess, jnp.int32(1), jnp.int32(0))

            accv = lax.fori_loop(0, _NBINS // 16, bbody, jnp.zeros((16,), jnp.int32))
            bstar = jnp.sum(accv)
            cb = plsc.load_gather(cbuf, [jnp.full((16,), bstar, jnp.int32)])
            g_above = n_match - jnp.max(cb)
            k_rem = k_rem - g_above
            prefix = lax.shift_left(prefix, 8) | bstar

        thresh = prefix ^ sign
        r_ties = k_rem

        def fbody(i, carry):
            c_eq, c_sel = carry
            key = keys[pl.ds(i * 16, 16)]
            gt = key > thresh
            eq = key == thresh
            eqi = jnp.where(eq, jnp.int32(1), jnp.int32(0))
            rank = plsc.cumsum(eqi) + c_eq
            sel = gt | (eq & (rank <= r_ties))
            seli = jnp.where(sel, jnp.int32(1), jnp.int32(0))
            mrow[pl.ds(i * 16, 16)] = seli
            pos = jnp.maximum(plsc.cumsum(seli) + (c_sel - jnp.int32(1)), jnp.int32(0))
            plsc.store_scatter(irow, [pos], lane + i * 16, mask=sel)
            return (c_eq + jnp.sum(eqi), c_sel + jnp.sum(seli))

        lax.fori_loop(0, _NV, fbody, (jnp.int32(0), jnp.int32(0)))

        pltpu.sync_copy(mrow, mask_hbm.at[pl.ds(row * _S, _S)])
        pltpu.sync_copy(irow, idx_hbm.at[pl.ds(row * _KPAD, _KPAD)])


@functools.partial(
    pl.kernel,
    mesh=plsc.VectorSubcoreMesh(core_axis_name="c", subcore_axis_name="s"),
    compiler_params=pltpu.CompilerParams(needs_layout_passes=False),
    out_type=[
        jax.ShapeDtypeStruct((_B * _S,), jnp.int32),
        jax.ShapeDtypeStruct((_B * _KPAD,), jnp.int32),
    ],
    scratch_types=[
        pltpu.VMEM((_S,), jnp.float32),
        pltpu.VMEM((_S,), jnp.int32),
        pltpu.VMEM((16 * _NBINS,), jnp.int32),
        pltpu.VMEM((_NBINS,), jnp.int32),
        pltpu.VMEM((_S,), jnp.int32),
        pltpu.VMEM((_KPAD,), jnp.int32),
    ],
)
def _select(logits_hbm, mask_hbm, idx_hbm, lrow, keys, hist, cbuf, mrow, irow):
    _select_body(logits_hbm, mask_hbm, idx_hbm, lrow, keys, hist, cbuf, mrow, irow)


def kernel(x, gate_w, gate_b, log_temp):
    x2d = x.reshape(_B * _S, _D)
    w = gate_w.reshape(_D, 1)
    b = gate_b.reshape(1, 1)
    logits2d, sp, ent = _gate(x2d, w, b)
    logits = logits2d.reshape(_B, _S)
    mask_i, idx_p = _select(logits2d.reshape(_B * _S))
    mask = mask_i.reshape(_B, _S).astype(jnp.bool_)
    indices = idx_p.reshape(_B, _KPAD)[:, :_K]
    mean_p = sp[0, 0] / (_B * _S)
    aux = 0.1 * (mean_p - 0.2) ** 2 + 0.01 * (ent[0, 0] / (_B * _S))
    return mask, indices, logits, aux


# speedup vs baseline: 1.2688x; 1.0364x over previous
import functools

import jax
import jax.numpy as jnp
from jax import lax
from jax.experimental import pallas as pl
from jax.experimental.pallas import tpu as pltpu
from jax.experimental.pallas import tpu_sc as plsc

_B, _S, _D = 4, 8192, 768
_K = 1638
_KPAD = 1664
_BLK = 2048
_NBLK = (_B * _S) // _BLK
_NV = _S // 16
_NBINS = 256


def _gate_body(x_ref, w_ref, b_ref, out_ref, sp_ref, ent_ref):
    i = pl.program_id(0)
    lg = lax.dot_general(
        x_ref[...], w_ref[...], (((1,), (0,)), ((), ())),
        preferred_element_type=jnp.float32,
    ) + b_ref[0, 0]
    out_ref[...] = lg
    p = jax.nn.sigmoid(lg)
    ent = -(p * jnp.log(p + 1e-10) + (1.0 - p) * jnp.log(1.0 - p + 1e-10))
    sp = jnp.sum(p)
    en = jnp.sum(ent)

    @pl.when(i == 0)
    def _init():
        sp_ref[0, 0] = sp
        ent_ref[0, 0] = en

    @pl.when(i != 0)
    def _acc():
        sp_ref[0, 0] += sp
        ent_ref[0, 0] += en


def _gate(x2d, w, b):
    return pl.pallas_call(
        _gate_body,
        grid=(_NBLK,),
        in_specs=[
            pl.BlockSpec((_BLK, _D), lambda i: (i, 0)),
            pl.BlockSpec((_D, 1), lambda i: (0, 0)),
            pl.BlockSpec(memory_space=pltpu.SMEM),
        ],
        out_specs=[
            pl.BlockSpec((_BLK, 1), lambda i: (i, 0)),
            pl.BlockSpec(memory_space=pltpu.SMEM),
            pl.BlockSpec(memory_space=pltpu.SMEM),
        ],
        out_shape=[
            jax.ShapeDtypeStruct((_B * _S, 1), jnp.float32),
            jax.ShapeDtypeStruct((1, 1), jnp.float32),
            jax.ShapeDtypeStruct((1, 1), jnp.float32),
        ],
        compiler_params=pltpu.CompilerParams(
            dimension_semantics=("arbitrary",),
        ),
    )(x2d, w, b)


def _select_body(logits_hbm, mask_hbm, idx_hbm,
                 lrow, keys, ck1, ck2, hist, cbuf, mrow, irow):
    wid = lax.axis_index("s") * 2 + lax.axis_index("c")

    @pl.when(wid < _B)
    def _():
        row = wid
        pltpu.sync_copy(logits_hbm.at[pl.ds(row * _S, _S)], lrow)

        lane = lax.iota(jnp.int32, 16)
        zeros16 = jnp.zeros((16,), jnp.int32)
        ones16 = jnp.ones((16,), jnp.int32)
        sign = jnp.int32(-2147483648)

        def zero_hist():
            def zbody(i, c):
                hist[pl.ds(i * 16, 16)] = zeros16
                return c
            lax.fori_loop(0, (16 * _NBINS) // 16, zbody, jnp.int32(0))

        def p1body(i, c):
            for u in range(4):
                j = i * 4 + u
                bits = lax.bitcast_convert_type(lrow[pl.ds(j * 16, 16)], jnp.int32)
                key = bits ^ (lax.shift_right_arithmetic(bits, 31) & jnp.int32(0x7FFFFFFF))
                key = key + jnp.where(key == jnp.int32(-1), jnp.int32(1), jnp.int32(0))
                keys[pl.ds(j * 16, 16)] = key
                binv = lax.shift_right_logical(key ^ sign, jnp.int32(24))
                plsc.addupdate_scatter(hist, [lane * _NBINS + binv], ones16)
            return c

        zero_hist()
        lax.fori_loop(0, _NV // 4, p1body, jnp.int32(0))

        def level_select(k_rem):
            def sbody(i, csum):
                tot = hist[pl.ds(i * 16, 16)]
                for j in range(1, 16):
                    tot = tot + hist[pl.ds(j * _NBINS + i * 16, 16)]
                cbuf[pl.ds(i * 16, 16)] = plsc.cumsum(tot) + csum
                return csum + jnp.sum(tot)

            n = lax.fori_loop(0, _NBINS // 16, sbody, jnp.int32(0))
            excess = n - k_rem

            def bbody(i, acc):
                c = cbuf[pl.ds(i * 16, 16)]
                return acc + jnp.where(c <= excess, jnp.int32(1), jnp.int32(0))

            accv = lax.fori_loop(0, _NBINS // 16, bbody, zeros16)
            bstar = jnp.sum(accv)
            cb = plsc.load_gather(cbuf, [jnp.full((16,), bstar, jnp.int32)])
            g_above = n - jnp.max(cb)
            return bstar, g_above

        def compact(src, dst, ntrips, nvalid, sh, bstar):
            nsplat = None if nvalid is None else jnp.full((16,), nvalid, jnp.int32)

            def cbody(i, off_vec):
                key = src[pl.ds(i * 16, 16)]
                binv = lax.shift_right_logical(key ^ sign, jnp.int32(sh)) & jnp.int32(_NBINS - 1)
                m = binv == bstar
                if nsplat is not None:
                    m = m & ((lane + i * 16) < nsplat)
                mi = jnp.where(m, jnp.int32(1), jnp.int32(0))
                pos = jnp.maximum(plsc.cumsum(mi) + off_vec, jnp.int32(0))
                plsc.store_scatter(dst, [pos], key, mask=m)
                return off_vec + plsc.all_reduce_population_count(m)

            off = lax.fori_loop(0, ntrips, cbody, jnp.full((16,), -1, jnp.int32))
            return jnp.max(off) + jnp.int32(1)

        def hist_pass(src, ntrips, nvalid, sh):
            nsplat = jnp.full((16,), nvalid, jnp.int32)

            def hbody(i, c):
                key = src[pl.ds(i * 16, 16)]
                binv = lax.shift_right_logical(key ^ sign, jnp.int32(sh)) & jnp.int32(_NBINS - 1)
                m = (lane + i * 16) < nsplat
                plsc.addupdate_scatter(hist, [lane * _NBINS + binv], ones16, mask=m)
                return c

            lax.fori_loop(0, ntrips, hbody, jnp.int32(0))

        k_rem = jnp.int32(_K)
        b0, g0 = level_select(k_rem)
        k_rem = k_rem - g0
        n1 = compact(keys, ck1, _NV, None, 24, b0)
        t1 = lax.div(n1 + jnp.int32(15), jnp.int32(16))

        zero_hist()
        hist_pass(ck1, t1, n1, 16)
        b1, g1 = level_select(k_rem)
        k_rem = k_rem - g1
        n2 = compact(ck1, ck2, t1, n1, 16, b1)
        t2 = lax.div(n2 + jnp.int32(15), jnp.int32(16))

        zero_hist()
        hist_pass(ck2, t2, n2, 8)
        b2, g2 = level_select(k_rem)
        k_rem = k_rem - g2
        n3 = compact(ck2, ck1, t2, n2, 8, b2)
        t3 = lax.div(n3 + jnp.int32(15), jnp.int32(16))

        zero_hist()
        hist_pass(ck1, t3, n3, 0)
        b3, g3 = level_select(k_rem)
        k_rem = k_rem - g3

        sl8 = jnp.int32(8)
        thresh = lax.shift_left(
            lax.shift_left(lax.shift_left(b0, sl8) | b1, sl8) | b2, sl8
        ) | b3
        thresh = thresh ^ sign
        r_ties = k_rem

        def fbody(i, carry):
            ceq, csel = carry
            for u in range(4):
                j = i * 4 + u
                key = keys[pl.ds(j * 16, 16)]
                gt = key > thresh
                eq = key == thresh
                eqi = jnp.where(eq, jnp.int32(1), jnp.int32(0))
                rank = plsc.cumsum(eqi) + ceq
                sel = gt | (eq & (rank <= r_ties))
                seli = jnp.where(sel, jnp.int32(1), jnp.int32(0))
                mrow[pl.ds(j * 16, 16)] = seli
                pos = jnp.maximum(plsc.cumsum(seli) + csel, jnp.int32(0))
                plsc.store_scatter(irow, [pos], lane + jnp.int32(j * 16), mask=sel)
                ceq = ceq + plsc.all_reduce_population_count(eq)
                csel = csel + plsc.all_reduce_population_count(sel)
            return ceq, csel

        lax.fori_loop(0, _NV // 4, fbody,
                      (zeros16, jnp.full((16,), -1, jnp.int32)))

        pltpu.sync_copy(mrow, mask_hbm.at[pl.ds(row * _S, _S)])
        pltpu.sync_copy(irow, idx_hbm.at[pl.ds(row * _KPAD, _KPAD)])


@functools.partial(
    pl.kernel,
    mesh=plsc.VectorSubcoreMesh(core_axis_name="c", subcore_axis_name="s"),
    compiler_params=pltpu.CompilerParams(needs_layout_passes=False),
    out_type=[
        jax.ShapeDtypeStruct((_B * _S,), jnp.int32),
        jax.ShapeDtypeStruct((_B * _KPAD,), jnp.int32),
    ],
    scratch_types=[
        pltpu.VMEM((_S,), jnp.float32),
        pltpu.VMEM((_S,), jnp.int32),
        pltpu.VMEM((_S,), jnp.int32),
        pltpu.VMEM((_S,), jnp.int32),
        pltpu.VMEM((16 * _NBINS,), jnp.int32),
        pltpu.VMEM((_NBINS,), jnp.int32),
        pltpu.VMEM((_S,), jnp.int32),
        pltpu.VMEM((_KPAD,), jnp.int32),
    ],
)
def _select(logits_hbm, mask_hbm, idx_hbm,
            lrow, keys, ck1, ck2, hist, cbuf, mrow, irow):
    _select_body(logits_hbm, mask_hbm, idx_hbm,
                 lrow, keys, ck1, ck2, hist, cbuf, mrow, irow)


def kernel(x, gate_w, gate_b, log_temp):
    x2d = x.reshape(_B * _S, _D)
    w = gate_w.reshape(_D, 1)
    b = gate_b.reshape(1, 1)
    logits2d, sp, ent = _gate(x2d, w, b)
    logits = logits2d.reshape(_B, _S)
    mask_i, idx_p = _select(logits2d.reshape(_B * _S))
    mask = mask_i.reshape(_B, _S).astype(jnp.bool_)
    indices = idx_p.reshape(_B, _KPAD)[:, :_K]
    mean_p = sp[0, 0] / (_B * _S)
    aux = 0.1 * (mean_p - 0.2) ** 2 + 0.01 * (ent[0, 0] / (_B * _S))
    return mask, indices, logits, aux


# speedup vs baseline: 1.5054x; 1.1864x over previous
import functools

import jax
import jax.numpy as jnp
from jax import lax
from jax.experimental import pallas as pl
from jax.experimental.pallas import tpu as pltpu
from jax.experimental.pallas import tpu_sc as plsc

_B, _S, _D = 4, 8192, 768
_K = 1638
_KPAD = 1664
_SEG = _KPAD // 8
_BLK = 2048
_NBLK = (_B * _S) // _BLK
_C = 1024
_NVC = _C // 16
_NBINS = 256


def _gate_body(x_ref, w_ref, b_ref, out_ref, sp_ref, ent_ref):
    i = pl.program_id(0)
    lg = lax.dot_general(
        x_ref[...], w_ref[...], (((1,), (0,)), ((), ())),
        preferred_element_type=jnp.float32,
    ) + b_ref[0, 0]
    out_ref[...] = lg
    p = jax.nn.sigmoid(lg)
    ent = -(p * jnp.log(p + 1e-10) + (1.0 - p) * jnp.log(1.0 - p + 1e-10))
    sp = jnp.sum(p)
    en = jnp.sum(ent)

    @pl.when(i == 0)
    def _init():
        sp_ref[0, 0] = sp
        ent_ref[0, 0] = en

    @pl.when(i != 0)
    def _acc():
        sp_ref[0, 0] += sp
        ent_ref[0, 0] += en


def _gate(x2d, w, b):
    return pl.pallas_call(
        _gate_body,
        grid=(_NBLK,),
        in_specs=[
            pl.BlockSpec((_BLK, _D), lambda i: (i, 0)),
            pl.BlockSpec((_D, 1), lambda i: (0, 0)),
            pl.BlockSpec(memory_space=pltpu.SMEM),
        ],
        out_specs=[
            pl.BlockSpec((_BLK, 1), lambda i: (i, 0)),
            pl.BlockSpec(memory_space=pltpu.SMEM),
            pl.BlockSpec(memory_space=pltpu.SMEM),
        ],
        out_shape=[
            jax.ShapeDtypeStruct((_B * _S, 1), jnp.float32),
            jax.ShapeDtypeStruct((1, 1), jnp.float32),
            jax.ShapeDtypeStruct((1, 1), jnp.float32),
        ],
        compiler_params=pltpu.CompilerParams(
            dimension_semantics=("arbitrary",),
        ),
    )(x2d, w, b)


def _select_body(logits_hbm, mask_hbm, idx_hbm,
                 lrow, keys, hist, cbuf, lck, ck1, ck2, asm,
                 mrow, ibuf, ibuf2, tmp, cnt,
                 sh_hist, sh_nck, sh_ck, sh_idx, sh_cnt, sh_pub):
    s = lax.axis_index("s")
    c = lax.axis_index("c")
    rl = s // 8
    cid = s % 8
    row = 2 * c + rl
    slot = rl * 8 + cid

    lane = lax.iota(jnp.int32, 16)
    zeros16 = jnp.zeros((16,), jnp.int32)
    ones16 = jnp.ones((16,), jnp.int32)
    sign = jnp.int32(-2147483648)

    def zero_hist():
        def zbody(i, cc):
            hist[pl.ds(i * 16, 16)] = zeros16
            return cc
        lax.fori_loop(0, (16 * _NBINS) // 16, zbody, jnp.int32(0))

    def splat(x):
        return jnp.full((16,), x, jnp.int32)

    pltpu.sync_copy(logits_hbm.at[pl.ds(row * _S + cid * _C, _C)], lrow)
    zero_hist()

    def p1body(i, cc):
        for u in range(4):
            j = i * 4 + u
            bits = lax.bitcast_convert_type(lrow[pl.ds(j * 16, 16)], jnp.int32)
            key = bits ^ (lax.shift_right_arithmetic(bits, 31) & jnp.int32(0x7FFFFFFF))
            key = key + jnp.where(key == jnp.int32(-1), jnp.int32(1), jnp.int32(0))
            keys[pl.ds(j * 16, 16)] = key
            binv = lax.shift_right_logical(key ^ sign, jnp.int32(24))
            plsc.addupdate_scatter(hist, [lane * _NBINS + binv], ones16)
        return cc

    lax.fori_loop(0, _NVC // 4, p1body, jnp.int32(0))

    def rbody(i, cc):
        tot = hist[pl.ds(i * 16, 16)]
        for j in range(1, 16):
            tot = tot + hist[pl.ds(j * _NBINS + i * 16, 16)]
        cbuf[pl.ds(i * 16, 16)] = tot
        return cc

    lax.fori_loop(0, _NBINS // 16, rbody, jnp.int32(0))
    pltpu.sync_copy(cbuf, sh_hist.at[pl.ds(slot * _NBINS, _NBINS)])
    plsc.subcore_barrier()

    def level_select_from_cbuf(n, k_rem):
        excess = n - k_rem

        def bbody(i, acc):
            cc = cbuf[pl.ds(i * 16, 16)]
            return acc + jnp.where(cc <= excess, jnp.int32(1), jnp.int32(0))

        accv = lax.fori_loop(0, _NBINS // 16, bbody, zeros16)
        bstar = jnp.sum(accv)
        cb = plsc.load_gather(cbuf, [splat(bstar)])
        g_above = n - jnp.max(cb)
        return bstar, g_above

    @pl.when(cid == 0)
    def _p2():
        pltpu.sync_copy(sh_hist.at[pl.ds(rl * 8 * _NBINS, 8 * _NBINS)],
                        asm.at[pl.ds(0, 8 * _NBINS)])

        def sbody(i, csum):
            tot = asm[pl.ds(i * 16, 16)]
            for j in range(1, 8):
                tot = tot + asm[pl.ds(j * _NBINS + i * 16, 16)]
            cbuf[pl.ds(i * 16, 16)] = plsc.cumsum(tot) + csum
            return csum + jnp.sum(tot)

        n0 = lax.fori_loop(0, _NBINS // 16, sbody, jnp.int32(0))
        b0, g0 = level_select_from_cbuf(n0, jnp.int32(_K))
        tmp[pl.ds(0, 16)] = splat(b0)
        tmp[pl.ds(16, 16)] = splat(g0)
        pltpu.sync_copy(tmp, sh_pub.at[pl.ds(rl * 32, 32)])

    plsc.subcore_barrier()

    pltpu.sync_copy(sh_pub.at[pl.ds(rl * 32, 32)], tmp)
    b0 = tmp[pl.ds(0, 16)][0]

    def c0body(i, off_vec):
        for u in range(4):
            j = i * 4 + u
            key = keys[pl.ds(j * 16, 16)]
            binv = lax.shift_right_logical(key ^ sign, jnp.int32(24))
            m = binv == b0
            mi = jnp.where(m, jnp.int32(1), jnp.int32(0))
            pos = jnp.maximum(plsc.cumsum(mi) + off_vec, jnp.int32(0))
            plsc.store_scatter(lck, [pos], key, mask=m)
            off_vec = off_vec + plsc.all_reduce_population_count(m)
        return off_vec

    offv = lax.fori_loop(0, _NVC // 4, c0body, jnp.full((16,), -1, jnp.int32))
    nck = jnp.max(offv) + jnp.int32(1)
    tmp[pl.ds(0, 16)] = splat(nck)
    pltpu.sync_copy(tmp.at[pl.ds(0, 16)], sh_nck.at[pl.ds(slot * 16, 16)])
    pltpu.sync_copy(lck, sh_ck.at[pl.ds(slot * _C, _C)])
    plsc.subcore_barrier()

    @pl.when(cid == 0)
    def _p4():
        pltpu.sync_copy(sh_ck.at[pl.ds(rl * 8 * _C, 8 * _C)], asm)
        pltpu.sync_copy(sh_nck.at[pl.ds(rl * 8 * 16, 8 * 16)],
                        cnt.at[pl.ds(0, 8 * 16)])
        g0 = tmp[pl.ds(16, 16)][0]
        k_rem = jnp.int32(_K) - g0

        off_vec = jnp.full((16,), -1, jnp.int32)
        for ch in range(8):
            n_ch = cnt[pl.ds(ch * 16, 16)][0]
            nspl = splat(n_ch)

            def abody(i, ov, ch=ch, nspl=nspl):
                m = (lane + i * 16) < nspl
                key = asm[pl.ds(ch * _C + i * 16, 16)]
                mi = jnp.where(m, jnp.int32(1), jnp.int32(0))
                pos = jnp.maximum(plsc.cumsum(mi) + ov, jnp.int32(0))
                plsc.store_scatter(ck1, [pos], key, mask=m)
                return ov + plsc.all_reduce_population_count(m)

            off_vec = lax.fori_loop(0, lax.div(n_ch + jnp.int32(15), jnp.int32(16)),
                                    abody, off_vec)
        n1 = jnp.max(off_vec) + jnp.int32(1)

        def hist_pass(src, ntrips, nvalid, sh):
            nspl = splat(nvalid)

            def hbody(i, cc):
                key = src[pl.ds(i * 16, 16)]
                binv = lax.shift_right_logical(key ^ sign, jnp.int32(sh)) & jnp.int32(_NBINS - 1)
                m = (lane + i * 16) < nspl
                plsc.addupdate_scatter(hist, [lane * _NBINS + binv], ones16, mask=m)
                return cc

            lax.fori_loop(0, ntrips, hbody, jnp.int32(0))

        def scan_hist(k_rem):
            def sbody(i, csum):
                tot = hist[pl.ds(i * 16, 16)]
                for j in range(1, 16):
                    tot = tot + hist[pl.ds(j * _NBINS + i * 16, 16)]
                cbuf[pl.ds(i * 16, 16)] = plsc.cumsum(tot) + csum
                return csum + jnp.sum(tot)

            n = lax.fori_loop(0, _NBINS // 16, sbody, jnp.int32(0))
            return level_select_from_cbuf(n, k_rem)

        def compact(src, dst, ntrips, nvalid, sh, bstar):
            nspl = splat(nvalid)

            def cbody(i, ov):
                key = src[pl.ds(i * 16, 16)]
                binv = lax.shift_right_logical(key ^ sign, jnp.int32(sh)) & jnp.int32(_NBINS - 1)
                m = (binv == bstar) & ((lane + i * 16) < nspl)
                mi = jnp.where(m, jnp.int32(1), jnp.int32(0))
                pos = jnp.maximum(plsc.cumsum(mi) + ov, jnp.int32(0))
                plsc.store_scatter(dst, [pos], key, mask=m)
                return ov + plsc.all_reduce_population_count(m)

            ov = lax.fori_loop(0, ntrips, cbody, jnp.full((16,), -1, jnp.int32))
            return jnp.max(ov) + jnp.int32(1)

        t1 = lax.div(n1 + jnp.int32(15), jnp.int32(16))
        zero_hist()
        hist_pass(ck1, t1, n1, 16)
        b1, g1 = scan_hist(k_rem)
        k_rem = k_rem - g1
        n2 = compact(ck1, ck2, t1, n1, 16, b1)
        t2 = lax.div(n2 + jnp.int32(15), jnp.int32(16))

        zero_hist()
        hist_pass(ck2, t2, n2, 8)
        b2, g2 = scan_hist(k_rem)
        k_rem = k_rem - g2
        n3 = compact(ck2, ck1, t2, n2, 8, b2)
        t3 = lax.div(n3 + jnp.int32(15), jnp.int32(16))

        zero_hist()
        hist_pass(ck1, t3, n3, 0)
        b3, g3 = scan_hist(k_rem)
        k_rem = k_rem - g3

        sl8 = jnp.int32(8)
        thresh = lax.shift_left(
            lax.shift_left(lax.shift_left(b0, sl8) | b1, sl8) | b2, sl8
        ) | b3
        thresh = thresh ^ sign
        tmp[pl.ds(0, 16)] = splat(thresh)
        tmp[pl.ds(16, 16)] = splat(k_rem)
        pltpu.sync_copy(tmp, sh_pub.at[pl.ds(rl * 32, 32)])

    plsc.subcore_barrier()

    pltpu.sync_copy(sh_pub.at[pl.ds(rl * 32, 32)], tmp)
    thresh = tmp[pl.ds(0, 16)][0]
    r_ties = tmp[pl.ds(16, 16)][0]

    def cntbody(i, carry):
        cgt, ceq = carry
        for u in range(4):
            j = i * 4 + u
            key = keys[pl.ds(j * 16, 16)]
            cgt = cgt + plsc.all_reduce_population_count(key > thresh)
            ceq = ceq + plsc.all_reduce_population_count(key == thresh)
        return cgt, ceq

    cgt, ceq = lax.fori_loop(0, _NVC // 4, cntbody, (zeros16, zeros16))
    tmp[pl.ds(0, 16)] = cgt
    tmp[pl.ds(16, 16)] = ceq
    pltpu.sync_copy(tmp, sh_cnt.at[pl.ds(slot * 32, 32)])
    plsc.subcore_barrier()

    pltpu.sync_copy(sh_cnt.at[pl.ds(rl * 8 * 32, 8 * 32)], cnt)
    zero = jnp.int32(0)
    eq_pref = zero
    sel_pref = zero
    sel_prefs = []
    my_eq_pref = zero
    my_sel_pref = zero
    for ch in range(8):
        sel_prefs.append(sel_pref)
        is_mine = cid == ch
        my_eq_pref = jnp.where(is_mine, eq_pref, my_eq_pref)
        my_sel_pref = jnp.where(is_mine, sel_pref, my_sel_pref)
        g_ch = cnt[pl.ds(ch * 32, 16)][0]
        e_ch = cnt[pl.ds(ch * 32 + 16, 16)][0]
        tie_ch = jnp.minimum(jnp.maximum(r_ties - eq_pref, zero), e_ch)
        eq_pref = eq_pref + e_ch
        sel_pref = sel_pref + g_ch + tie_ch

    def fbody(i, carry):
        ceqv, cselv = carry
        for u in range(4):
            j = i * 4 + u
            key = keys[pl.ds(j * 16, 16)]
            gt = key > thresh
            eq = key == thresh
            eqi = jnp.where(eq, jnp.int32(1), jnp.int32(0))
            rank = plsc.cumsum(eqi) + ceqv
            sel = gt | (eq & (rank <= r_ties))
            seli = jnp.where(sel, jnp.int32(1), jnp.int32(0))
            mrow[pl.ds(j * 16, 16)] = seli
            pos = jnp.maximum(plsc.cumsum(seli) + cselv, jnp.int32(0))
            plsc.store_scatter(ibuf, [pos], lane + jnp.int32(j * 16) + cid * _C, mask=sel)
            ceqv = ceqv + plsc.all_reduce_population_count(eq)
            cselv = cselv + plsc.all_reduce_population_count(sel)
        return ceqv, cselv

    lax.fori_loop(0, _NVC // 4, fbody,
                  (jnp.full((16,), my_eq_pref, jnp.int32),
                   jnp.full((16,), -1, jnp.int32)))

    pltpu.sync_copy(mrow, mask_hbm.at[pl.ds(row * _S + cid * _C, _C)])
    pltpu.sync_copy(ibuf, sh_idx.at[pl.ds(slot * _C, _C)])
    plsc.subcore_barrier()

    pltpu.sync_copy(sh_idx.at[pl.ds(rl * 8 * _C, 8 * _C)], asm)
    for v in range(_SEG // 16):
        p = splat(cid * _SEG + v * 16) + lane
        sc = zeros16
        for ch in range(1, 8):
            sc = sc + jnp.where(p >= splat(sel_prefs[ch]), jnp.int32(1), jnp.int32(0))
        off = p
        for ch in range(8):
            off = jnp.where(sc == ch, p - sel_prefs[ch] + jnp.int32(ch * _C), off)
        off = jnp.minimum(jnp.maximum(off, jnp.int32(0)), jnp.int32(8 * _C - 1))
        ibuf2[pl.ds(v * 16, 16)] = plsc.load_gather(asm, [off])

    pltpu.sync_copy(ibuf2, idx_hbm.at[pl.ds(row * _KPAD + cid * _SEG, _SEG)])


@functools.partial(
    pl.kernel,
    mesh=plsc.VectorSubcoreMesh(core_axis_name="c", subcore_axis_name="s"),
    compiler_params=pltpu.CompilerParams(needs_layout_passes=False),
    out_type=[
        jax.ShapeDtypeStruct((_B * _S,), jnp.int32),
        jax.ShapeDtypeStruct((_B * _KPAD,), jnp.int32),
    ],
    scratch_types=[
        pltpu.VMEM((_C,), jnp.float32),
        pltpu.VMEM((_C,), jnp.int32),
        pltpu.VMEM((16 * _NBINS,), jnp.int32),
        pltpu.VMEM((_NBINS,), jnp.int32),
        pltpu.VMEM((_C,), jnp.int32),
        pltpu.VMEM((8 * _C,), jnp.int32),
        pltpu.VMEM((8 * _C,), jnp.int32),
        pltpu.VMEM((8 * _C,), jnp.int32),
        pltpu.VMEM((_C,), jnp.int32),
        pltpu.VMEM((_C,), jnp.int32),
        pltpu.VMEM((_SEG,), jnp.int32),
        pltpu.VMEM((32,), jnp.int32),
        pltpu.VMEM((8 * 32,), jnp.int32),
        pltpu.VMEM_SHARED((2 * 8 * _NBINS,), jnp.int32),
        pltpu.VMEM_SHARED((2 * 8 * 16,), jnp.int32),
        pltpu.VMEM_SHARED((2 * 8 * _C,), jnp.int32),
        pltpu.VMEM_SHARED((2 * 8 * _C,), jnp.int32),
        pltpu.VMEM_SHARED((2 * 8 * 32,), jnp.int32),
        pltpu.VMEM_SHARED((2 * 32,), jnp.int32),
    ],
)
def _select(logits_hbm, mask_hbm, idx_hbm,
            lrow, keys, hist, cbuf, lck, ck1, ck2, asm,
            mrow, ibuf, ibuf2, tmp, cnt,
            sh_hist, sh_nck, sh_ck, sh_idx, sh_cnt, sh_pub):
    _select_body(logits_hbm, mask_hbm, idx_hbm,
                 lrow, keys, hist, cbuf, lck, ck1, ck2, asm,
                 mrow, ibuf, ibuf2, tmp, cnt,
                 sh_hist, sh_nck, sh_ck, sh_idx, sh_cnt, sh_pub)


def kernel(x, gate_w, gate_b, log_temp):
    x2d = x.reshape(_B * _S, _D)
    w = gate_w.reshape(_D, 1)
    b = gate_b.reshape(1, 1)
    logits2d, sp, ent = _gate(x2d, w, b)
    logits = logits2d.reshape(_B, _S)
    mask_i, idx_p = _select(logits2d.reshape(_B * _S))
    mask = mask_i.reshape(_B, _S).astype(jnp.bool_)
    indices = idx_p.reshape(_B, _KPAD)[:, :_K]
    mean_p = sp[0, 0] / (_B * _S)
    aux = 0.1 * (mean_p - 0.2) ** 2 + 0.01 * (ent[0, 0] / (_B * _S))
    return mask, indices, logits, aux


# speedup vs baseline: 1.5866x; 1.0539x over previous
import functools

import jax
import jax.numpy as jnp
from jax import lax
from jax.experimental import pallas as pl
from jax.experimental.pallas import tpu as pltpu
from jax.experimental.pallas import tpu_sc as plsc

_B, _S, _D = 4, 8192, 768
_K = 1638
_KPAD = 1664
_SEG = _KPAD // 8
_BLK = 4096
_NBLK = (_B * _S) // _BLK
_C = 1024
_NVC = _C // 16
_NBINS = 256


def _gate_body(x_ref, w_ref, b_ref, out_ref, sp_ref, ent_ref):
    i = pl.program_id(0)
    lg = lax.dot_general(
        x_ref[...], w_ref[...], (((1,), (0,)), ((), ())),
        preferred_element_type=jnp.float32,
    ) + b_ref[0, 0]
    out_ref[...] = lg
    p = jax.nn.sigmoid(lg)
    ent = -(p * jnp.log(p + 1e-10) + (1.0 - p) * jnp.log(1.0 - p + 1e-10))
    sp = jnp.sum(p)
    en = jnp.sum(ent)

    @pl.when(i == 0)
    def _init():
        sp_ref[0, 0] = sp
        ent_ref[0, 0] = en

    @pl.when(i != 0)
    def _acc():
        sp_ref[0, 0] += sp
        ent_ref[0, 0] += en


def _gate(x2d, w, b):
    return pl.pallas_call(
        _gate_body,
        grid=(_NBLK,),
        in_specs=[
            pl.BlockSpec((_BLK, _D), lambda i: (i, 0)),
            pl.BlockSpec((_D, 1), lambda i: (0, 0)),
            pl.BlockSpec(memory_space=pltpu.SMEM),
        ],
        out_specs=[
            pl.BlockSpec((_BLK, 1), lambda i: (i, 0)),
            pl.BlockSpec(memory_space=pltpu.SMEM),
            pl.BlockSpec(memory_space=pltpu.SMEM),
        ],
        out_shape=[
            jax.ShapeDtypeStruct((_B * _S, 1), jnp.float32),
            jax.ShapeDtypeStruct((1, 1), jnp.float32),
            jax.ShapeDtypeStruct((1, 1), jnp.float32),
        ],
        compiler_params=pltpu.CompilerParams(
            dimension_semantics=("arbitrary",),
        ),
    )(x2d, w, b)


def _select_body(logits_hbm, mask_hbm, idx_hbm,
                 lrow, keys, hist, cbuf, lck, ck1, ck2, asm,
                 mrow, ibuf, ibuf2, tmp, cnt,
                 sh_hist, sh_nck, sh_ck, sh_idx, sh_cnt):
    s = lax.axis_index("s")
    c = lax.axis_index("c")
    rl = s // 8
    cid = s % 8
    row = 2 * c + rl
    slot = rl * 8 + cid

    lane = lax.iota(jnp.int32, 16)
    zeros16 = jnp.zeros((16,), jnp.int32)
    ones16 = jnp.ones((16,), jnp.int32)
    sign = jnp.int32(-2147483648)

    def zero_hist():
        def zbody(i, cc):
            hist[pl.ds(i * 16, 16)] = zeros16
            return cc
        lax.fori_loop(0, (16 * _NBINS) // 16, zbody, jnp.int32(0))

    def splat(x):
        return jnp.full((16,), x, jnp.int32)

    pltpu.sync_copy(logits_hbm.at[pl.ds(row * _S + cid * _C, _C)], lrow)
    zero_hist()

    def p1body(i, cc):
        for u in range(4):
            j = i * 4 + u
            bits = lax.bitcast_convert_type(lrow[pl.ds(j * 16, 16)], jnp.int32)
            key = bits ^ (lax.shift_right_arithmetic(bits, 31) & jnp.int32(0x7FFFFFFF))
            key = key + jnp.where(key == jnp.int32(-1), jnp.int32(1), jnp.int32(0))
            keys[pl.ds(j * 16, 16)] = key
            binv = lax.shift_right_logical(key ^ sign, jnp.int32(24))
            plsc.addupdate_scatter(hist, [lane * _NBINS + binv], ones16)
        return cc

    lax.fori_loop(0, _NVC // 4, p1body, jnp.int32(0))

    def rbody(i, cc):
        tot = hist[pl.ds(i * 16, 16)]
        for j in range(1, 16):
            tot = tot + hist[pl.ds(j * _NBINS + i * 16, 16)]
        cbuf[pl.ds(i * 16, 16)] = tot
        return cc

    lax.fori_loop(0, _NBINS // 16, rbody, jnp.int32(0))
    pltpu.sync_copy(cbuf, sh_hist.at[pl.ds(slot * _NBINS, _NBINS)])
    plsc.subcore_barrier()

    def level_select_from_cbuf(n, k_rem):
        excess = n - k_rem

        def bbody(i, acc):
            cc = cbuf[pl.ds(i * 16, 16)]
            return acc + jnp.where(cc <= excess, jnp.int32(1), jnp.int32(0))

        accv = lax.fori_loop(0, _NBINS // 16, bbody, zeros16)
        bstar = jnp.sum(accv)
        cb = plsc.load_gather(cbuf, [splat(bstar)])
        g_above = n - jnp.max(cb)
        return bstar, g_above

    pltpu.sync_copy(sh_hist.at[pl.ds(rl * 8 * _NBINS, 8 * _NBINS)],
                    asm.at[pl.ds(0, 8 * _NBINS)])

    def sbody0(i, csum):
        tot = asm[pl.ds(i * 16, 16)]
        for j in range(1, 8):
            tot = tot + asm[pl.ds(j * _NBINS + i * 16, 16)]
        cbuf[pl.ds(i * 16, 16)] = plsc.cumsum(tot) + csum
        return csum + jnp.sum(tot)

    n0 = lax.fori_loop(0, _NBINS // 16, sbody0, jnp.int32(0))
    b0, g0 = level_select_from_cbuf(n0, jnp.int32(_K))


    def c0body(i, off_vec):
        for u in range(4):
            j = i * 4 + u
            key = keys[pl.ds(j * 16, 16)]
            binv = lax.shift_right_logical(key ^ sign, jnp.int32(24))
            m = binv == b0
            mi = jnp.where(m, jnp.int32(1), jnp.int32(0))
            pos = jnp.maximum(plsc.cumsum(mi) + off_vec, jnp.int32(0))
            plsc.store_scatter(lck, [pos], key, mask=m)
            off_vec = off_vec + plsc.all_reduce_population_count(m)
        return off_vec

    offv = lax.fori_loop(0, _NVC // 4, c0body, jnp.full((16,), -1, jnp.int32))
    nck = jnp.max(offv) + jnp.int32(1)
    tmp[pl.ds(0, 16)] = splat(nck)
    pltpu.sync_copy(tmp.at[pl.ds(0, 16)], sh_nck.at[pl.ds(slot * 16, 16)])
    pltpu.sync_copy(lck, sh_ck.at[pl.ds(slot * _C, _C)])
    plsc.subcore_barrier()

    pltpu.sync_copy(sh_ck.at[pl.ds(rl * 8 * _C, 8 * _C)], asm)
    pltpu.sync_copy(sh_nck.at[pl.ds(rl * 8 * 16, 8 * 16)],
                    cnt.at[pl.ds(0, 8 * 16)])
    k_rem = jnp.int32(_K) - g0

    off_vec = jnp.full((16,), -1, jnp.int32)
    for ch in range(8):
        n_ch = cnt[pl.ds(ch * 16, 16)][0]
        nspl = splat(n_ch)

        def abody(i, ov, ch=ch, nspl=nspl):
            m = (lane + i * 16) < nspl
            key = asm[pl.ds(ch * _C + i * 16, 16)]
            mi = jnp.where(m, jnp.int32(1), jnp.int32(0))
            pos = jnp.maximum(plsc.cumsum(mi) + ov, jnp.int32(0))
            plsc.store_scatter(ck1, [pos], key, mask=m)
            return ov + plsc.all_reduce_population_count(m)

        off_vec = lax.fori_loop(0, lax.div(n_ch + jnp.int32(15), jnp.int32(16)),
                                abody, off_vec)
    n1 = jnp.max(off_vec) + jnp.int32(1)

    def hist_pass(src, ntrips, nvalid, sh):
        nspl = splat(nvalid)

        def hbody(i, cc):
            key = src[pl.ds(i * 16, 16)]
            binv = lax.shift_right_logical(key ^ sign, jnp.int32(sh)) & jnp.int32(_NBINS - 1)
            m = (lane + i * 16) < nspl
            plsc.addupdate_scatter(hist, [lane * _NBINS + binv], ones16, mask=m)
            return cc

        lax.fori_loop(0, ntrips, hbody, jnp.int32(0))

    def zero_scatter(src, ntrips, nvalid, sh):
        nspl = splat(nvalid)

        def zbody(i, cc):
            key = src[pl.ds(i * 16, 16)]
            binv = lax.shift_right_logical(key ^ sign, jnp.int32(sh)) & jnp.int32(_NBINS - 1)
            m = (lane + i * 16) < nspl
            plsc.store_scatter(hist, [lane * _NBINS + binv], zeros16, mask=m)
            return cc

        lax.fori_loop(0, ntrips, zbody, jnp.int32(0))

    def scan_hist(k_rem):
        def sbody(i, csum):
            tot = hist[pl.ds(i * 16, 16)]
            for j in range(1, 16):
                tot = tot + hist[pl.ds(j * _NBINS + i * 16, 16)]
            cbuf[pl.ds(i * 16, 16)] = plsc.cumsum(tot) + csum
            return csum + jnp.sum(tot)

        n = lax.fori_loop(0, _NBINS // 16, sbody, jnp.int32(0))
        return level_select_from_cbuf(n, k_rem)

    def compact(src, dst, ntrips, nvalid, sh, bstar):
        nspl = splat(nvalid)

        def cbody(i, ov):
            key = src[pl.ds(i * 16, 16)]
            binv = lax.shift_right_logical(key ^ sign, jnp.int32(sh)) & jnp.int32(_NBINS - 1)
            m = (binv == bstar) & ((lane + i * 16) < nspl)
            mi = jnp.where(m, jnp.int32(1), jnp.int32(0))
            pos = jnp.maximum(plsc.cumsum(mi) + ov, jnp.int32(0))
            plsc.store_scatter(dst, [pos], key, mask=m)
            return ov + plsc.all_reduce_population_count(m)

        ov = lax.fori_loop(0, ntrips, cbody, jnp.full((16,), -1, jnp.int32))
        return jnp.max(ov) + jnp.int32(1)

    t1 = lax.div(n1 + jnp.int32(15), jnp.int32(16))
    zero_hist()
    hist_pass(ck1, t1, n1, 16)
    b1, g1 = scan_hist(k_rem)
    k_rem = k_rem - g1
    n2 = compact(ck1, ck2, t1, n1, 16, b1)
    t2 = lax.div(n2 + jnp.int32(15), jnp.int32(16))

    zero_scatter(ck1, t1, n1, 16)
    hist_pass(ck2, t2, n2, 8)
    b2, g2 = scan_hist(k_rem)
    k_rem = k_rem - g2
    n3 = compact(ck2, ck1, t2, n2, 8, b2)
    t3 = lax.div(n3 + jnp.int32(15), jnp.int32(16))

    zero_scatter(ck2, t2, n2, 8)
    hist_pass(ck1, t3, n3, 0)
    b3, g3 = scan_hist(k_rem)
    k_rem = k_rem - g3

    sl8 = jnp.int32(8)
    thresh = lax.shift_left(
        lax.shift_left(lax.shift_left(b0, sl8) | b1, sl8) | b2, sl8
    ) | b3
    thresh = thresh ^ sign
    r_ties = k_rem


    def cntbody(i, carry):
        cgt, ceq = carry
        for u in range(4):
            j = i * 4 + u
            key = keys[pl.ds(j * 16, 16)]
            cgt = cgt + plsc.all_reduce_population_count(key > thresh)
            ceq = ceq + plsc.all_reduce_population_count(key == thresh)
        return cgt, ceq

    cgt, ceq = lax.fori_loop(0, _NVC // 4, cntbody, (zeros16, zeros16))
    tmp[pl.ds(0, 16)] = cgt
    tmp[pl.ds(16, 16)] = ceq
    pltpu.sync_copy(tmp, sh_cnt.at[pl.ds(slot * 32, 32)])
    plsc.subcore_barrier()

    pltpu.sync_copy(sh_cnt.at[pl.ds(rl * 8 * 32, 8 * 32)], cnt)
    zero = jnp.int32(0)
    eq_pref = zero
    sel_pref = zero
    sel_prefs = []
    my_eq_pref = zero
    my_sel_pref = zero
    for ch in range(8):
        sel_prefs.append(sel_pref)
        is_mine = cid == ch
        my_eq_pref = jnp.where(is_mine, eq_pref, my_eq_pref)
        my_sel_pref = jnp.where(is_mine, sel_pref, my_sel_pref)
        g_ch = cnt[pl.ds(ch * 32, 16)][0]
        e_ch = cnt[pl.ds(ch * 32 + 16, 16)][0]
        tie_ch = jnp.minimum(jnp.maximum(r_ties - eq_pref, zero), e_ch)
        eq_pref = eq_pref + e_ch
        sel_pref = sel_pref + g_ch + tie_ch

    def fbody(i, carry):
        ceqv, cselv = carry
        for u in range(4):
            j = i * 4 + u
            key = keys[pl.ds(j * 16, 16)]
            gt = key > thresh
            eq = key == thresh
            eqi = jnp.where(eq, jnp.int32(1), jnp.int32(0))
            rank = plsc.cumsum(eqi) + ceqv
            sel = gt | (eq & (rank <= r_ties))
            seli = jnp.where(sel, jnp.int32(1), jnp.int32(0))
            mrow[pl.ds(j * 16, 16)] = seli
            pos = jnp.maximum(plsc.cumsum(seli) + cselv, jnp.int32(0))
            plsc.store_scatter(ibuf, [pos], lane + jnp.int32(j * 16) + cid * _C, mask=sel)
            ceqv = ceqv + plsc.all_reduce_population_count(eq)
            cselv = cselv + plsc.all_reduce_population_count(sel)
        return ceqv, cselv

    lax.fori_loop(0, _NVC // 4, fbody,
                  (jnp.full((16,), my_eq_pref, jnp.int32),
                   jnp.full((16,), -1, jnp.int32)))

    pltpu.sync_copy(mrow, mask_hbm.at[pl.ds(row * _S + cid * _C, _C)])
    pltpu.sync_copy(ibuf, sh_idx.at[pl.ds(slot * _C, _C)])
    plsc.subcore_barrier()

    pltpu.sync_copy(sh_idx.at[pl.ds(rl * 8 * _C, 8 * _C)], asm)
    for v in range(_SEG // 16):
        p = splat(cid * _SEG + v * 16) + lane
        sc = zeros16
        for ch in range(1, 8):
            sc = sc + jnp.where(p >= splat(sel_prefs[ch]), jnp.int32(1), jnp.int32(0))
        off = p
        for ch in range(8):
            off = jnp.where(sc == ch, p - sel_prefs[ch] + jnp.int32(ch * _C), off)
        off = jnp.minimum(jnp.maximum(off, jnp.int32(0)), jnp.int32(8 * _C - 1))
        ibuf2[pl.ds(v * 16, 16)] = plsc.load_gather(asm, [off])

    pltpu.sync_copy(ibuf2, idx_hbm.at[pl.ds(row * _KPAD + cid * _SEG, _SEG)])


@functools.partial(
    pl.kernel,
    mesh=plsc.VectorSubcoreMesh(core_axis_name="c", subcore_axis_name="s"),
    compiler_params=pltpu.CompilerParams(needs_layout_passes=False),
    out_type=[
        jax.ShapeDtypeStruct((_B * _S,), jnp.int32),
        jax.ShapeDtypeStruct((_B * _KPAD,), jnp.int32),
    ],
    scratch_types=[
        pltpu.VMEM((_C,), jnp.float32),
        pltpu.VMEM((_C,), jnp.int32),
        pltpu.VMEM((16 * _NBINS,), jnp.int32),
        pltpu.VMEM((_NBINS,), jnp.int32),
        pltpu.VMEM((_C,), jnp.int32),
        pltpu.VMEM((8 * _C,), jnp.int32),
        pltpu.VMEM((8 * _C,), jnp.int32),
        pltpu.VMEM((8 * _C,), jnp.int32),
        pltpu.VMEM((_C,), jnp.int32),
        pltpu.VMEM((_C,), jnp.int32),
        pltpu.VMEM((_SEG,), jnp.int32),
        pltpu.VMEM((32,), jnp.int32),
        pltpu.VMEM((8 * 32,), jnp.int32),
        pltpu.VMEM_SHARED((2 * 8 * _NBINS,), jnp.int32),
        pltpu.VMEM_SHARED((2 * 8 * 16,), jnp.int32),
        pltpu.VMEM_SHARED((2 * 8 * _C,), jnp.int32),
        pltpu.VMEM_SHARED((2 * 8 * _C,), jnp.int32),
        pltpu.VMEM_SHARED((2 * 8 * 32,), jnp.int32),
    ],
)
def _select(logits_hbm, mask_hbm, idx_hbm,
            lrow, keys, hist, cbuf, lck, ck1, ck2, asm,
            mrow, ibuf, ibuf2, tmp, cnt,
            sh_hist, sh_nck, sh_ck, sh_idx, sh_cnt):
    _select_body(logits_hbm, mask_hbm, idx_hbm,
                 lrow, keys, hist, cbuf, lck, ck1, ck2, asm,
                 mrow, ibuf, ibuf2, tmp, cnt,
                 sh_hist, sh_nck, sh_ck, sh_idx, sh_cnt)


def kernel(x, gate_w, gate_b, log_temp):
    x2d = x.reshape(_B * _S, _D)
    w = gate_w.reshape(_D, 1)
    b = gate_b.reshape(1, 1)
    logits2d, sp, ent = _gate(x2d, w, b)
    logits = logits2d.reshape(_B, _S)
    mask_i, idx_p = _select(logits2d.reshape(_B * _S))
    mask = mask_i.reshape(_B, _S).astype(jnp.bool_)
    indices = idx_p.reshape(_B, _KPAD)[:, :_K]
    mean_p = sp[0, 0] / (_B * _S)
    aux = 0.1 * (mean_p - 0.2) ** 2 + 0.01 * (ent[0, 0] / (_B * _S))
    return mask, indices, logits, aux
